# Initial kernel scaffold; baseline (speedup 1.0000x reference)
import jax
import jax.numpy as jnp
from jax.experimental import pallas as pl


def kernel(edge_index, real_features, cat_features, mask, emb_table, W_gcn, b_gcn, W1, b1, W2, b2):
    raise NotImplementedError("write your pallas kernel here")


# speedup vs baseline: 36.3354x; 36.3354x over previous
import functools

import jax
import jax.numpy as jnp
from jax import lax
from jax.experimental import pallas as pl
from jax.experimental.pallas import tpu as pltpu
from jax.experimental.pallas import tpu_sc as plsc

N = 100000
E = 6400000
D = 16
DEG_COL = 10

NC = 2
NS = 16
NW = NC * NS
EPW = E // NW

CH = 80
KG = 10
GROUP = CH * KG
NGROUPS = EPW // GROUP

K = 50000
KP = 51200
KPW = KP // NW
CH3 = 80
NCH3 = KPW // CH3


_BLK0 = 1000


def _build_x_body(real_ref, cat_ref, emb_ref, out_ref):
    cat = cat_ref[:]
    onehot = (lax.broadcasted_iota(jnp.int32, (_BLK0, 6), 1) == cat)
    emb = jnp.dot(onehot.astype(jnp.float32), emb_ref[:],
                  preferred_element_type=jnp.float32)
    out_ref[:] = jnp.concatenate(
        [real_ref[:], emb,
         jnp.ones((_BLK0, 1), jnp.float32),
         jnp.zeros((_BLK0, D - DEG_COL - 1), jnp.float32)], axis=1)


def _build_x(real, cat, emb_table):
    return pl.pallas_call(
        _build_x_body,
        grid=(N // _BLK0,),
        in_specs=[
            pl.BlockSpec((_BLK0, 5), lambda i: (i, 0)),
            pl.BlockSpec((_BLK0, 1), lambda i: (i, 0)),
            pl.BlockSpec((6, 5), lambda i: (0, 0)),
        ],
        out_specs=pl.BlockSpec((_BLK0, D), lambda i: (i, 0)),
        out_shape=jax.ShapeDtypeStruct((N, D), jnp.float32),
    )(real, cat, emb_table)


_MESH = plsc.VectorSubcoreMesh(core_axis_name="c", subcore_axis_name="s")
_SC_PARAMS = pltpu.CompilerParams(use_tc_tiling_on_sc=False)


@functools.partial(
    pl.kernel,
    out_type=jax.ShapeDtypeStruct((NC, NS, N // NS, D), jnp.float32),
    mesh=_MESH,
    compiler_params=_SC_PARAMS,
    scratch_types=[
        pltpu.VMEM((GROUP,), jnp.int32),
        pltpu.VMEM((KG, CH), jnp.int32),
        pltpu.VMEM((GROUP, D), jnp.float32),
        pltpu.VMEM_SHARED((N, D), jnp.float32),
        pltpu.SemaphoreType.DMA,
    ],
)
def _sc_aggregate(src_hbm, dst2_hbm, x_hbm, zeros_hbm, out_hbm,
                  sidx, didx, rows, acc, gsem):
    c = lax.axis_index("c")
    s = lax.axis_index("s")
    wid = c * NS + s
    rpt = N // NS

    pltpu.sync_copy(zeros_hbm.at[s], acc.at[pl.ds(s * rpt, rpt)])
    plsc.subcore_barrier()

    base = wid * EPW

    @pl.loop(0, NGROUPS)
    def _group(g):
        off = base + g * GROUP
        grow = wid * NGROUPS + g
        pltpu.sync_copy(src_hbm.at[pl.ds(off, GROUP)], sidx)
        pltpu.sync_copy(dst2_hbm.at[grow], didx)
        descs = []
        for j in range(KG):
            descs.append(pltpu.async_copy(
                x_hbm.at[sidx.at[pl.ds(j * CH, CH)]],
                rows.at[pl.ds(j * CH, CH)], gsem))
        for d in descs:
            d.wait()
        for j in range(KG):
            pltpu.sync_copy(rows.at[pl.ds(j * CH, CH)],
                            acc.at[didx.at[j]], add=True)

    plsc.subcore_barrier()
    pltpu.sync_copy(acc.at[pl.ds(s * rpt, rpt)], out_hbm.at[c, s])


_BLK2 = 1000


def _mlp_body(agg_ref, w0_ref, b0_ref, w1_ref, b1_ref, w2_ref, b2_ref,
              out_ref):
    t = agg_ref[0] + agg_ref[1]
    deg = jnp.maximum(t[:, DEG_COL:DEG_COL + 1], 1.0)
    sm = t / deg
    h = jnp.maximum(jnp.dot(sm, w0_ref[:], preferred_element_type=jnp.float32)
                    + b0_ref[:], 0.0)
    h = jnp.maximum(jnp.dot(h, w1_ref[:], preferred_element_type=jnp.float32)
                    + b1_ref[:], 0.0)
    lo = jnp.dot(h, w2_ref[:], preferred_element_type=jnp.float32) + b2_ref[:]
    out_ref[:] = lo + jnp.zeros((1, 8), jnp.float32)


def _mlp_head(aggs, w0, b0, w1, b1, w2, b2):
    full = lambda shape: pl.BlockSpec(shape, lambda i: tuple(0 for _ in shape))
    return pl.pallas_call(
        _mlp_body,
        grid=(N // _BLK2,),
        in_specs=[
            pl.BlockSpec((2, _BLK2, D), lambda i: (0, i, 0)),
            full((D, 16)), full((1, 16)),
            full((16, 24)), full((1, 24)),
            full((24, 1)), full((1, 1)),
        ],
        out_specs=pl.BlockSpec((_BLK2, 8), lambda i: (i, 0)),
        out_shape=jax.ShapeDtypeStruct((N, 8), jnp.float32),
    )(aggs, w0, b0, w1, b1, w2, b2)


@functools.partial(
    pl.kernel,
    out_type=jax.ShapeDtypeStruct((KP, 8), jnp.float32),
    mesh=_MESH,
    compiler_params=_SC_PARAMS,
    scratch_types=[
        pltpu.VMEM((KPW,), jnp.int32),
        pltpu.VMEM((KPW, 8), jnp.float32),
        pltpu.SemaphoreType.DMA,
    ],
)
def _sc_gather(logits_hbm, mask_hbm, out_hbm, midx, vals, sem):
    c = lax.axis_index("c")
    s = lax.axis_index("s")
    wid = c * NS + s
    base = wid * KPW
    pltpu.sync_copy(mask_hbm.at[pl.ds(base, KPW)], midx)
    descs = []
    for j in range(NCH3):
        descs.append(pltpu.async_copy(
            logits_hbm.at[midx.at[pl.ds(j * CH3, CH3)]],
            vals.at[pl.ds(j * CH3, CH3)], sem))
    for d in descs:
        d.wait()
    pltpu.sync_copy(vals, out_hbm.at[pl.ds(base, KPW)])


_R4 = KP // 128


def _softmax_body(sel_ref, out_ref):
    x = sel_ref[:]
    flat_idx = (lax.broadcasted_iota(jnp.int32, (_R4, 128), 0) * 128
                + lax.broadcasted_iota(jnp.int32, (_R4, 128), 1))
    valid = flat_idx < K
    neg = jnp.full_like(x, -jnp.inf)
    m = jnp.max(jnp.where(valid, x, neg))
    e = jnp.where(valid, jnp.exp(x - m), 0.0)
    out_ref[:] = e / jnp.sum(e)


def _softmax(sel2):
    return pl.pallas_call(
        _softmax_body,
        out_shape=jax.ShapeDtypeStruct((_R4, 128), jnp.float32),
    )(sel2)


def kernel(edge_index, real_features, cat_features, mask, emb_table,
           W_gcn, b_gcn, W1, b1, W2, b2):
    src = edge_index[0]
    dst2 = edge_index[1].reshape(E // GROUP, KG, CH)

    x_aug = _build_x(real_features, cat_features, emb_table)

    zeros = jnp.zeros((NS, N // NS, D), jnp.float32)
    aggs = _sc_aggregate(src, dst2, x_aug, zeros).reshape(NC, N, D)

    w0 = jnp.zeros((D, 16), jnp.float32).at[:DEG_COL].set(W_gcn)
    logits = _mlp_head(aggs, w0, b_gcn.reshape(1, 16),
                       W1, b1.reshape(1, 24), W2, b2.reshape(1, 1))

    mask_p = jnp.concatenate([mask, jnp.zeros((KP - K,), jnp.int32)])
    sel = _sc_gather(logits, mask_p)

    probs = _softmax(sel[:, 0].reshape(_R4, 128))
    return probs.reshape(-1)[:K]


# speedup vs baseline: 57.0103x; 1.5690x over previous
import functools

import jax
import jax.numpy as jnp
from jax import lax
from jax.experimental import pallas as pl
from jax.experimental.pallas import tpu as pltpu
from jax.experimental.pallas import tpu_sc as plsc

N = 100000
E = 6400000
D = 16
DEG_COL = 10

NC = 2
NS = 16
NW = NC * NS
EPW = E // NW

CH = 80
KG = 10
GROUP = CH * KG
NGROUPS = EPW // GROUP

K = 50000
KP = 51200
KPW = KP // NW
CH3 = 80
NCH3 = KPW // CH3


_BLK0 = 1000


def _build_x_body(real_ref, cat_ref, emb_ref, out_ref):
    cat = cat_ref[:]
    onehot = (lax.broadcasted_iota(jnp.int32, (_BLK0, 6), 1) == cat)
    emb = jnp.dot(onehot.astype(jnp.float32), emb_ref[:],
                  preferred_element_type=jnp.float32)
    out_ref[:] = jnp.concatenate(
        [real_ref[:], emb,
         jnp.ones((_BLK0, 1), jnp.float32),
         jnp.zeros((_BLK0, D - DEG_COL - 1), jnp.float32)], axis=1)


def _build_x(real, cat, emb_table):
    return pl.pallas_call(
        _build_x_body,
        grid=(N // _BLK0,),
        in_specs=[
            pl.BlockSpec((_BLK0, 5), lambda i: (i, 0)),
            pl.BlockSpec((_BLK0, 1), lambda i: (i, 0)),
            pl.BlockSpec((6, 5), lambda i: (0, 0)),
        ],
        out_specs=pl.BlockSpec((_BLK0, D), lambda i: (i, 0)),
        out_shape=jax.ShapeDtypeStruct((N, D), jnp.float32),
    )(real, cat, emb_table)


_MESH = plsc.VectorSubcoreMesh(core_axis_name="c", subcore_axis_name="s")
_SC_PARAMS = pltpu.CompilerParams(use_tc_tiling_on_sc=False)


@functools.partial(
    pl.kernel,
    out_type=jax.ShapeDtypeStruct((NC, NS, N // NS, D), jnp.float32),
    mesh=_MESH,
    compiler_params=_SC_PARAMS,
    scratch_types=[
        pltpu.VMEM((2, GROUP), jnp.int32),
        pltpu.VMEM((2, KG, CH), jnp.int32),
        pltpu.VMEM((2, GROUP, D), jnp.float32),
        pltpu.VMEM_SHARED((N, D), jnp.float32),
        pltpu.SemaphoreType.DMA,
        pltpu.SemaphoreType.DMA,
        pltpu.SemaphoreType.DMA,
    ],
)
def _sc_aggregate(src_hbm, dst2_hbm, x_hbm, zeros_hbm, out_hbm,
                  sidx, didx, rows, acc, gsem, ssem, isem):
    c = lax.axis_index("c")
    s = lax.axis_index("s")
    wid = c * NS + s
    rpt = N // NS

    pltpu.sync_copy(zeros_hbm.at[s], acc.at[pl.ds(s * rpt, rpt)])
    plsc.subcore_barrier()

    base = wid * EPW

    pltpu.async_copy(src_hbm.at[pl.ds(base, GROUP)], sidx.at[0], isem)
    pltpu.async_copy(dst2_hbm.at[wid * NGROUPS], didx.at[0], isem)

    @pl.loop(0, NGROUPS)
    def _group(g):
        cur = lax.rem(g, 2)
        nxt = 1 - cur

        @pl.when(g >= 2)
        def _():
            pltpu.make_async_copy(
                x_hbm.at[pl.ds(0, GROUP)], rows.at[cur], ssem).wait()

        pltpu.make_async_copy(
            src_hbm.at[pl.ds(0, GROUP)], sidx.at[cur], isem).wait()
        pltpu.make_async_copy(dst2_hbm.at[0], didx.at[cur], isem).wait()

        descs = []
        for j in range(KG):
            descs.append(pltpu.async_copy(
                x_hbm.at[sidx.at[cur, pl.ds(j * CH, CH)]],
                rows.at[cur, pl.ds(j * CH, CH)], gsem))

        @pl.when(g + 1 < NGROUPS)
        def _():
            off = base + (g + 1) * GROUP
            pltpu.async_copy(src_hbm.at[pl.ds(off, GROUP)], sidx.at[nxt],
                             isem)
            pltpu.async_copy(dst2_hbm.at[wid * NGROUPS + g + 1],
                             didx.at[nxt], isem)

        for d_ in descs:
            d_.wait()

        for j in range(KG):
            pltpu.async_copy(rows.at[cur, pl.ds(j * CH, CH)],
                             acc.at[didx.at[cur, j]], ssem, add=True)

    pltpu.make_async_copy(x_hbm.at[pl.ds(0, GROUP)], rows.at[0], ssem).wait()
    pltpu.make_async_copy(x_hbm.at[pl.ds(0, GROUP)], rows.at[1], ssem).wait()

    plsc.subcore_barrier()
    pltpu.sync_copy(acc.at[pl.ds(s * rpt, rpt)], out_hbm.at[c, s])


_BLK2 = 1000


def _mlp_body(agg_ref, w0_ref, b0_ref, w1_ref, b1_ref, w2_ref, b2_ref,
              out_ref):
    t = agg_ref[0] + agg_ref[1]
    deg = jnp.maximum(t[:, DEG_COL:DEG_COL + 1], 1.0)
    sm = t / deg
    h = jnp.maximum(jnp.dot(sm, w0_ref[:], preferred_element_type=jnp.float32)
                    + b0_ref[:], 0.0)
    h = jnp.maximum(jnp.dot(h, w1_ref[:], preferred_element_type=jnp.float32)
                    + b1_ref[:], 0.0)
    lo = jnp.dot(h, w2_ref[:], preferred_element_type=jnp.float32) + b2_ref[:]
    out_ref[:] = lo + jnp.zeros((1, 8), jnp.float32)


def _mlp_head(aggs, w0, b0, w1, b1, w2, b2):
    full = lambda shape: pl.BlockSpec(shape, lambda i: tuple(0 for _ in shape))
    return pl.pallas_call(
        _mlp_body,
        grid=(N // _BLK2,),
        in_specs=[
            pl.BlockSpec((2, _BLK2, D), lambda i: (0, i, 0)),
            full((D, 16)), full((1, 16)),
            full((16, 24)), full((1, 24)),
            full((24, 1)), full((1, 1)),
        ],
        out_specs=pl.BlockSpec((_BLK2, 8), lambda i: (i, 0)),
        out_shape=jax.ShapeDtypeStruct((N, 8), jnp.float32),
    )(aggs, w0, b0, w1, b1, w2, b2)


@functools.partial(
    pl.kernel,
    out_type=jax.ShapeDtypeStruct((KP, 8), jnp.float32),
    mesh=_MESH,
    compiler_params=_SC_PARAMS,
    scratch_types=[
        pltpu.VMEM((KPW,), jnp.int32),
        pltpu.VMEM((KPW, 8), jnp.float32),
        pltpu.SemaphoreType.DMA,
    ],
)
def _sc_gather(logits_hbm, mask_hbm, out_hbm, midx, vals, sem):
    c = lax.axis_index("c")
    s = lax.axis_index("s")
    wid = c * NS + s
    base = wid * KPW
    pltpu.sync_copy(mask_hbm.at[pl.ds(base, KPW)], midx)
    descs = []
    for j in range(NCH3):
        descs.append(pltpu.async_copy(
            logits_hbm.at[midx.at[pl.ds(j * CH3, CH3)]],
            vals.at[pl.ds(j * CH3, CH3)], sem))
    for d in descs:
        d.wait()
    pltpu.sync_copy(vals, out_hbm.at[pl.ds(base, KPW)])


_R4 = KP // 128


def _softmax_body(sel_ref, out_ref):
    x = sel_ref[:]
    flat_idx = (lax.broadcasted_iota(jnp.int32, (_R4, 128), 0) * 128
                + lax.broadcasted_iota(jnp.int32, (_R4, 128), 1))
    valid = flat_idx < K
    neg = jnp.full_like(x, -jnp.inf)
    m = jnp.max(jnp.where(valid, x, neg))
    e = jnp.where(valid, jnp.exp(x - m), 0.0)
    out_ref[:] = e / jnp.sum(e)


def _softmax(sel2):
    return pl.pallas_call(
        _softmax_body,
        out_shape=jax.ShapeDtypeStruct((_R4, 128), jnp.float32),
    )(sel2)


def kernel(edge_index, real_features, cat_features, mask, emb_table,
           W_gcn, b_gcn, W1, b1, W2, b2):
    src = edge_index[0]
    dst2 = edge_index[1].reshape(E // GROUP, KG, CH)

    x_aug = _build_x(real_features, cat_features, emb_table)

    zeros = jnp.zeros((NS, N // NS, D), jnp.float32)
    aggs = _sc_aggregate(src, dst2, x_aug, zeros).reshape(NC, N, D)

    w0 = jnp.zeros((D, 16), jnp.float32).at[:DEG_COL].set(W_gcn)
    logits = _mlp_head(aggs, w0, b_gcn.reshape(1, 16),
                       W1, b1.reshape(1, 24), W2, b2.reshape(1, 1))

    mask_p = jnp.concatenate([mask, jnp.zeros((KP - K,), jnp.int32)])
    sel = _sc_gather(logits, mask_p)

    probs = _softmax(sel[:, 0].reshape(_R4, 128))
    return probs.reshape(-1)[:K]


# speedup vs baseline: 57.0309x; 1.0004x over previous
import functools

import jax
import jax.numpy as jnp
from jax import lax
from jax.experimental import pallas as pl
from jax.experimental.pallas import tpu as pltpu
from jax.experimental.pallas import tpu_sc as plsc

N = 100000
E = 6400000
D = 16
DEG_COL = 10

NC = 2
NS = 16
NW = NC * NS
EPW = E // NW

CH = 80
KG = 10
GROUP = CH * KG
NGROUPS = EPW // GROUP

K = 50000
KP = 51200
KPW = KP // NW
CH3 = 80
NCH3 = KPW // CH3


_BLK0 = 1000


def _build_x_body(real_ref, cat_ref, emb_ref, out_ref):
    cat = cat_ref[:]
    onehot = (lax.broadcasted_iota(jnp.int32, (_BLK0, 6), 1) == cat)
    emb = jnp.dot(onehot.astype(jnp.float32), emb_ref[:],
                  preferred_element_type=jnp.float32)
    out_ref[:] = jnp.concatenate(
        [real_ref[:], emb,
         jnp.ones((_BLK0, 1), jnp.float32),
         jnp.zeros((_BLK0, D - DEG_COL - 1), jnp.float32)], axis=1)


def _build_x(real, cat, emb_table):
    return pl.pallas_call(
        _build_x_body,
        grid=(N // _BLK0,),
        in_specs=[
            pl.BlockSpec((_BLK0, 5), lambda i: (i, 0)),
            pl.BlockSpec((_BLK0, 1), lambda i: (i, 0)),
            pl.BlockSpec((6, 5), lambda i: (0, 0)),
        ],
        out_specs=pl.BlockSpec((_BLK0, D), lambda i: (i, 0)),
        out_shape=jax.ShapeDtypeStruct((N, D), jnp.float32),
    )(real, cat, emb_table)


_MESH = plsc.VectorSubcoreMesh(core_axis_name="c", subcore_axis_name="s")
_SC_PARAMS = pltpu.CompilerParams(use_tc_tiling_on_sc=False)


@functools.partial(
    pl.kernel,
    out_type=jax.ShapeDtypeStruct((NC, NS, N // NS, D), jnp.float32),
    mesh=_MESH,
    compiler_params=_SC_PARAMS,
    scratch_types=[
        pltpu.VMEM((2, GROUP), jnp.int32),
        pltpu.VMEM((3, KG, CH), jnp.int32),
        pltpu.VMEM((2, GROUP, D), jnp.float32),
        pltpu.VMEM_SHARED((N, D), jnp.float32),
        pltpu.SemaphoreType.DMA,
        pltpu.SemaphoreType.DMA,
        pltpu.SemaphoreType.DMA,
    ],
)
def _sc_aggregate(src_hbm, dst2_hbm, x_hbm, zeros_hbm, out_hbm,
                  sidx, didx, rows, acc, gsem, ssem, isem):
    c = lax.axis_index("c")
    s = lax.axis_index("s")
    wid = c * NS + s
    rpt = N // NS

    pltpu.sync_copy(zeros_hbm.at[s], acc.at[pl.ds(s * rpt, rpt)])
    plsc.subcore_barrier()

    base = wid * EPW

    pltpu.async_copy(src_hbm.at[pl.ds(base, GROUP)], sidx.at[0], isem)
    pltpu.async_copy(dst2_hbm.at[wid * NGROUPS], didx.at[0], isem)

    @pl.loop(0, NGROUPS)
    def _group(g):
        cur = lax.rem(g, 2)
        nxt = 1 - cur
        cur3 = lax.rem(g, 3)
        nxt3 = lax.rem(g + 1, 3)

        @pl.when(g >= 2)
        def _():
            pltpu.make_async_copy(
                x_hbm.at[pl.ds(0, GROUP)], rows.at[cur], ssem).wait()

        pltpu.make_async_copy(
            src_hbm.at[pl.ds(0, GROUP)], sidx.at[cur], isem).wait()
        pltpu.make_async_copy(dst2_hbm.at[0], didx.at[cur3], isem).wait()

        descs = []
        for j in range(KG):
            descs.append(pltpu.async_copy(
                x_hbm.at[sidx.at[cur, pl.ds(j * CH, CH)]],
                rows.at[cur, pl.ds(j * CH, CH)], gsem))

        @pl.when(g + 1 < NGROUPS)
        def _():
            off = base + (g + 1) * GROUP
            pltpu.async_copy(src_hbm.at[pl.ds(off, GROUP)], sidx.at[nxt],
                             isem)
            pltpu.async_copy(dst2_hbm.at[wid * NGROUPS + g + 1],
                             didx.at[nxt3], isem)

        for d_ in descs:
            d_.wait()

        for j in range(KG):
            pltpu.async_copy(rows.at[cur, pl.ds(j * CH, CH)],
                             acc.at[didx.at[cur3, j]], ssem, add=True)

    pltpu.make_async_copy(x_hbm.at[pl.ds(0, GROUP)], rows.at[0], ssem).wait()
    pltpu.make_async_copy(x_hbm.at[pl.ds(0, GROUP)], rows.at[1], ssem).wait()

    plsc.subcore_barrier()
    pltpu.sync_copy(acc.at[pl.ds(s * rpt, rpt)], out_hbm.at[c, s])


_BLK2 = 1000


def _mlp_body(agg_ref, w0_ref, b0_ref, w1_ref, b1_ref, w2_ref, b2_ref,
              out_ref):
    t = agg_ref[0] + agg_ref[1]
    deg = jnp.maximum(t[:, DEG_COL:DEG_COL + 1], 1.0)
    sm = t / deg
    h = jnp.maximum(jnp.dot(sm, w0_ref[:], preferred_element_type=jnp.float32)
                    + b0_ref[:], 0.0)
    h = jnp.maximum(jnp.dot(h, w1_ref[:], preferred_element_type=jnp.float32)
                    + b1_ref[:], 0.0)
    lo = jnp.dot(h, w2_ref[:], preferred_element_type=jnp.float32) + b2_ref[:]
    out_ref[:] = lo + jnp.zeros((1, 8), jnp.float32)


def _mlp_head(aggs, w0, b0, w1, b1, w2, b2):
    full = lambda shape: pl.BlockSpec(shape, lambda i: tuple(0 for _ in shape))
    return pl.pallas_call(
        _mlp_body,
        grid=(N // _BLK2,),
        in_specs=[
            pl.BlockSpec((2, _BLK2, D), lambda i: (0, i, 0)),
            full((D, 16)), full((1, 16)),
            full((16, 24)), full((1, 24)),
            full((24, 1)), full((1, 1)),
        ],
        out_specs=pl.BlockSpec((_BLK2, 8), lambda i: (i, 0)),
        out_shape=jax.ShapeDtypeStruct((N, 8), jnp.float32),
    )(aggs, w0, b0, w1, b1, w2, b2)


@functools.partial(
    pl.kernel,
    out_type=jax.ShapeDtypeStruct((KP, 8), jnp.float32),
    mesh=_MESH,
    compiler_params=_SC_PARAMS,
    scratch_types=[
        pltpu.VMEM((KPW,), jnp.int32),
        pltpu.VMEM((KPW, 8), jnp.float32),
        pltpu.SemaphoreType.DMA,
    ],
)
def _sc_gather(logits_hbm, mask_hbm, out_hbm, midx, vals, sem):
    c = lax.axis_index("c")
    s = lax.axis_index("s")
    wid = c * NS + s
    base = wid * KPW
    pltpu.sync_copy(mask_hbm.at[pl.ds(base, KPW)], midx)
    descs = []
    for j in range(NCH3):
        descs.append(pltpu.async_copy(
            logits_hbm.at[midx.at[pl.ds(j * CH3, CH3)]],
            vals.at[pl.ds(j * CH3, CH3)], sem))
    for d in descs:
        d.wait()
    pltpu.sync_copy(vals, out_hbm.at[pl.ds(base, KPW)])


_R4 = KP // 128


def _softmax_body(sel_ref, out_ref):
    x = sel_ref[:]
    flat_idx = (lax.broadcasted_iota(jnp.int32, (_R4, 128), 0) * 128
                + lax.broadcasted_iota(jnp.int32, (_R4, 128), 1))
    valid = flat_idx < K
    neg = jnp.full_like(x, -jnp.inf)
    m = jnp.max(jnp.where(valid, x, neg))
    e = jnp.where(valid, jnp.exp(x - m), 0.0)
    out_ref[:] = e / jnp.sum(e)


def _softmax(sel2):
    return pl.pallas_call(
        _softmax_body,
        out_shape=jax.ShapeDtypeStruct((_R4, 128), jnp.float32),
    )(sel2)


def kernel(edge_index, real_features, cat_features, mask, emb_table,
           W_gcn, b_gcn, W1, b1, W2, b2):
    src = edge_index[0]
    dst2 = edge_index[1].reshape(E // GROUP, KG, CH)

    x_aug = _build_x(real_features, cat_features, emb_table)

    zeros = jnp.zeros((NS, N // NS, D), jnp.float32)
    aggs = _sc_aggregate(src, dst2, x_aug, zeros).reshape(NC, N, D)

    w0 = jnp.zeros((D, 16), jnp.float32).at[:DEG_COL].set(W_gcn)
    logits = _mlp_head(aggs, w0, b_gcn.reshape(1, 16),
                       W1, b1.reshape(1, 24), W2, b2.reshape(1, 1))

    mask_p = jnp.concatenate([mask, jnp.zeros((KP - K,), jnp.int32)])
    sel = _sc_gather(logits, mask_p)

    probs = _softmax(sel[:, 0].reshape(_R4, 128))
    return probs.reshape(-1)[:K]


# speedup vs baseline: 65.8376x; 1.1544x over previous
import functools

import jax
import jax.numpy as jnp
from jax import lax
from jax.experimental import pallas as pl
from jax.experimental.pallas import tpu as pltpu
from jax.experimental.pallas import tpu_sc as plsc

N = 100000
E = 6400000
D = 16
DEG_COL = 10

NC = 2
NS = 16
NW = NC * NS
EPW = E // NW

CH = 80
KG = 10
GROUP = CH * KG
NGROUPS = EPW // GROUP

K = 50000
KP = 51200
KPW = KP // NW
CH3 = 80
NCH3 = KPW // CH3


_BLK0 = 5000


def _build_x_body(real_ref, cat_ref, emb_ref, out_ref):
    cat = cat_ref[:]
    onehot = (lax.broadcasted_iota(jnp.int32, (_BLK0, 6), 1) == cat)
    emb = jnp.dot(onehot.astype(jnp.float32), emb_ref[:],
                  preferred_element_type=jnp.float32)
    out_ref[:] = jnp.concatenate(
        [real_ref[:], emb,
         jnp.ones((_BLK0, 1), jnp.float32),
         jnp.zeros((_BLK0, D - DEG_COL - 1), jnp.float32)], axis=1)


def _build_x(real, cat, emb_table):
    return pl.pallas_call(
        _build_x_body,
        grid=(N // _BLK0,),
        in_specs=[
            pl.BlockSpec((_BLK0, 5), lambda i: (i, 0)),
            pl.BlockSpec((_BLK0, 1), lambda i: (i, 0)),
            pl.BlockSpec((6, 5), lambda i: (0, 0)),
        ],
        out_specs=pl.BlockSpec((_BLK0, D), lambda i: (i, 0)),
        out_shape=jax.ShapeDtypeStruct((N, D), jnp.float32),
    )(real, cat, emb_table)


_MESH = plsc.VectorSubcoreMesh(core_axis_name="c", subcore_axis_name="s")
_SC_PARAMS = pltpu.CompilerParams(use_tc_tiling_on_sc=False)


@functools.partial(
    pl.kernel,
    out_type=jax.ShapeDtypeStruct((NC, N, D), jnp.float32),
    mesh=_MESH,
    compiler_params=_SC_PARAMS,
    scratch_types=[
        pltpu.VMEM((2, GROUP), jnp.int32),
        pltpu.VMEM((3, GROUP), jnp.int32),
        pltpu.VMEM((2, GROUP, D), jnp.float32),
        pltpu.VMEM_SHARED((N, D), jnp.float32),
        pltpu.SemaphoreType.DMA,
        pltpu.SemaphoreType.DMA,
        pltpu.SemaphoreType.DMA,
    ],
)
def _sc_aggregate(edges_hbm, x_hbm, zeros_hbm, out_hbm,
                  sidx, didx, rows, acc, gsem, ssem, isem):
    c = lax.axis_index("c")
    s = lax.axis_index("s")
    wid = c * NS + s
    rpt = N // NS

    pltpu.sync_copy(zeros_hbm.at[s], acc.at[pl.ds(s * rpt, rpt)])
    plsc.subcore_barrier()

    base = wid * EPW

    pltpu.async_copy(edges_hbm.at[0, pl.ds(base, GROUP)], sidx.at[0], isem)
    pltpu.async_copy(edges_hbm.at[1, pl.ds(base, GROUP)], didx.at[0], isem)

    @pl.loop(0, NGROUPS)
    def _group(g):
        cur = lax.rem(g, 2)
        nxt = 1 - cur
        cur3 = lax.rem(g, 3)
        nxt3 = lax.rem(g + 1, 3)

        @pl.when(g >= 2)
        def _():
            pltpu.make_async_copy(
                x_hbm.at[pl.ds(0, GROUP)], rows.at[cur], ssem).wait()

        pltpu.make_async_copy(
            edges_hbm.at[0, pl.ds(0, GROUP)], sidx.at[cur], isem).wait()
        pltpu.make_async_copy(
            edges_hbm.at[1, pl.ds(0, GROUP)], didx.at[cur3], isem).wait()

        descs = []
        for j in range(KG):
            descs.append(pltpu.async_copy(
                x_hbm.at[sidx.at[cur, pl.ds(j * CH, CH)]],
                rows.at[cur, pl.ds(j * CH, CH)], gsem))

        @pl.when(g + 1 < NGROUPS)
        def _():
            off = base + (g + 1) * GROUP
            pltpu.async_copy(edges_hbm.at[0, pl.ds(off, GROUP)],
                             sidx.at[nxt], isem)
            pltpu.async_copy(edges_hbm.at[1, pl.ds(off, GROUP)],
                             didx.at[nxt3], isem)

        for d_ in descs:
            d_.wait()

        for j in range(KG):
            pltpu.async_copy(rows.at[cur, pl.ds(j * CH, CH)],
                             acc.at[didx.at[cur3, pl.ds(j * CH, CH)]],
                             ssem, add=True)

    pltpu.make_async_copy(x_hbm.at[pl.ds(0, GROUP)], rows.at[0], ssem).wait()
    pltpu.make_async_copy(x_hbm.at[pl.ds(0, GROUP)], rows.at[1], ssem).wait()

    plsc.subcore_barrier()
    pltpu.sync_copy(acc.at[pl.ds(s * rpt, rpt)],
                    out_hbm.at[c, pl.ds(s * rpt, rpt)])


_BLK2 = 5000


def _mlp_body(agg_ref, w0_ref, b0_ref, w1_ref, b1_ref, w2_ref, b2_ref,
              out_ref):
    t = agg_ref[0] + agg_ref[1]
    deg = jnp.maximum(t[:, DEG_COL:DEG_COL + 1], 1.0)
    sm = t / deg
    h = jnp.maximum(jnp.dot(sm, w0_ref[:], preferred_element_type=jnp.float32)
                    + b0_ref[:], 0.0)
    h = jnp.maximum(jnp.dot(h, w1_ref[:], preferred_element_type=jnp.float32)
                    + b1_ref[:], 0.0)
    lo = jnp.dot(h, w2_ref[:], preferred_element_type=jnp.float32) + b2_ref[:]
    out_ref[:] = lo + jnp.zeros((1, 8), jnp.float32)


def _mlp_head(aggs, w0, b0, w1, b1, w2, b2):
    full = lambda shape: pl.BlockSpec(shape, lambda i: tuple(0 for _ in shape))
    return pl.pallas_call(
        _mlp_body,
        grid=(N // _BLK2,),
        in_specs=[
            pl.BlockSpec((2, _BLK2, D), lambda i: (0, i, 0)),
            full((D, 16)), full((1, 16)),
            full((16, 24)), full((1, 24)),
            full((24, 1)), full((1, 1)),
        ],
        out_specs=pl.BlockSpec((_BLK2, 8), lambda i: (i, 0)),
        out_shape=jax.ShapeDtypeStruct((N, 8), jnp.float32),
    )(aggs, w0, b0, w1, b1, w2, b2)


@functools.partial(
    pl.kernel,
    out_type=jax.ShapeDtypeStruct((KP, 8), jnp.float32),
    mesh=_MESH,
    compiler_params=_SC_PARAMS,
    scratch_types=[
        pltpu.VMEM((KPW,), jnp.int32),
        pltpu.VMEM((KPW, 8), jnp.float32),
        pltpu.SemaphoreType.DMA,
    ],
)
def _sc_gather(logits_hbm, mask_hbm, out_hbm, midx, vals, sem):
    c = lax.axis_index("c")
    s = lax.axis_index("s")
    wid = c * NS + s
    base = wid * KPW
    pltpu.sync_copy(mask_hbm.at[pl.ds(base, KPW)], midx)
    descs = []
    for j in range(NCH3):
        descs.append(pltpu.async_copy(
            logits_hbm.at[midx.at[pl.ds(j * CH3, CH3)]],
            vals.at[pl.ds(j * CH3, CH3)], sem))
    for d in descs:
        d.wait()
    pltpu.sync_copy(vals, out_hbm.at[pl.ds(base, KPW)])


_R4 = KP // 128


def _softmax_body(sel_ref, out_ref):
    x = sel_ref[:]
    flat_idx = (lax.broadcasted_iota(jnp.int32, (_R4, 128), 0) * 128
                + lax.broadcasted_iota(jnp.int32, (_R4, 128), 1))
    valid = flat_idx < K
    neg = jnp.full_like(x, -jnp.inf)
    m = jnp.max(jnp.where(valid, x, neg))
    e = jnp.where(valid, jnp.exp(x - m), 0.0)
    out_ref[:] = e / jnp.sum(e)


def _softmax(sel2):
    return pl.pallas_call(
        _softmax_body,
        out_shape=jax.ShapeDtypeStruct((_R4, 128), jnp.float32),
    )(sel2)


def kernel(edge_index, real_features, cat_features, mask, emb_table,
           W_gcn, b_gcn, W1, b1, W2, b2):
    x_aug = _build_x(real_features, cat_features, emb_table)

    zeros = jnp.zeros((NS, N // NS, D), jnp.float32)
    aggs = _sc_aggregate(edge_index, x_aug, zeros)

    w0 = jnp.zeros((D, 16), jnp.float32).at[:DEG_COL].set(W_gcn)
    logits = _mlp_head(aggs, w0, b_gcn.reshape(1, 16),
                       W1, b1.reshape(1, 24), W2, b2.reshape(1, 1))

    mask_p = jnp.concatenate([mask, jnp.zeros((KP - K,), jnp.int32)])
    sel = _sc_gather(logits, mask_p)

    probs = _softmax(sel[:, 0].reshape(_R4, 128))
    return probs.reshape(-1)[:K]


# speedup vs baseline: 75.2770x; 1.1434x over previous
import functools

import jax
import jax.numpy as jnp
from jax import lax
from jax.experimental import pallas as pl
from jax.experimental.pallas import tpu as pltpu
from jax.experimental.pallas import tpu_sc as plsc

N = 100000
E = 6400000
D = 16
DEG_COL = 10

NC = 2
NS = 16
NW = NC * NS
EPW = E // NW

CH = 80
KG = 10
GROUP = CH * KG
NGROUPS = EPW // GROUP

K = 50000
KP = 51200
KPW = KP // NW
CH3 = 80
NCH3 = KPW // CH3


_BLK0 = 5000


def _build_x_body(real_ref, cat_ref, emb_ref, out_ref):
    cat = cat_ref[:]
    onehot = (lax.broadcasted_iota(jnp.int32, (_BLK0, 6), 1) == cat)
    emb = jnp.dot(onehot.astype(jnp.float32), emb_ref[:],
                  preferred_element_type=jnp.float32)
    out_ref[:] = jnp.concatenate(
        [real_ref[:], emb,
         jnp.ones((_BLK0, 1), jnp.float32),
         jnp.zeros((_BLK0, D - DEG_COL - 1), jnp.float32)], axis=1)


def _build_x(real, cat, emb_table):
    return pl.pallas_call(
        _build_x_body,
        grid=(N // _BLK0,),
        in_specs=[
            pl.BlockSpec((_BLK0, 5), lambda i: (i, 0)),
            pl.BlockSpec((_BLK0, 1), lambda i: (i, 0)),
            pl.BlockSpec((6, 5), lambda i: (0, 0)),
        ],
        out_specs=pl.BlockSpec((_BLK0, D), lambda i: (i, 0)),
        out_shape=jax.ShapeDtypeStruct((N, D), jnp.float32),
    )(real, cat, emb_table)


_MESH = plsc.VectorSubcoreMesh(core_axis_name="c", subcore_axis_name="s")
_SC_PARAMS = pltpu.CompilerParams(use_tc_tiling_on_sc=False)


@functools.partial(
    pl.kernel,
    out_type=jax.ShapeDtypeStruct((NC, N, D), jnp.float32),
    mesh=_MESH,
    compiler_params=_SC_PARAMS,
    scratch_types=[
        pltpu.VMEM((2, GROUP), jnp.int32),
        pltpu.VMEM((3, GROUP), jnp.int32),
        pltpu.VMEM((2, GROUP, D), jnp.float32),
        pltpu.VMEM_SHARED((N, D), jnp.float32),
        pltpu.SemaphoreType.DMA,
        pltpu.SemaphoreType.DMA,
        pltpu.SemaphoreType.DMA,
    ],
)
def _sc_aggregate(edges_hbm, x_hbm, zeros_hbm, out_hbm,
                  sidx, didx, rows, acc, gsem, ssem, isem):
    c = lax.axis_index("c")
    s = lax.axis_index("s")
    wid = c * NS + s
    rpt = N // NS

    pltpu.sync_copy(zeros_hbm.at[s], acc.at[pl.ds(s * rpt, rpt)])
    plsc.subcore_barrier()

    base = wid * EPW

    pltpu.async_copy(edges_hbm.at[0, pl.ds(base, GROUP)], sidx.at[0], isem)
    pltpu.async_copy(edges_hbm.at[1, pl.ds(base, GROUP)], didx.at[0], isem)

    @pl.loop(0, NGROUPS)
    def _group(g):
        cur = lax.rem(g, 2)
        nxt = 1 - cur
        cur3 = lax.rem(g, 3)
        nxt3 = lax.rem(g + 1, 3)

        @pl.when(g >= 2)
        def _():
            pltpu.make_async_copy(
                x_hbm.at[pl.ds(0, GROUP)], rows.at[cur], ssem).wait()

        pltpu.make_async_copy(
            edges_hbm.at[0, pl.ds(0, GROUP)], sidx.at[cur], isem).wait()
        pltpu.make_async_copy(
            edges_hbm.at[1, pl.ds(0, GROUP)], didx.at[cur3], isem).wait()

        descs = []
        for j in range(KG):
            descs.append(pltpu.async_copy(
                x_hbm.at[sidx.at[cur, pl.ds(j * CH, CH)]],
                rows.at[cur, pl.ds(j * CH, CH)], gsem))

        @pl.when(g + 1 < NGROUPS)
        def _():
            off = base + (g + 1) * GROUP
            pltpu.async_copy(edges_hbm.at[0, pl.ds(off, GROUP)],
                             sidx.at[nxt], isem)
            pltpu.async_copy(edges_hbm.at[1, pl.ds(off, GROUP)],
                             didx.at[nxt3], isem)

        for d_ in descs:
            d_.wait()

        for j in range(KG):
            pltpu.async_copy(rows.at[cur, pl.ds(j * CH, CH)],
                             acc.at[didx.at[cur3, pl.ds(j * CH, CH)]],
                             ssem, add=True)

    pltpu.make_async_copy(x_hbm.at[pl.ds(0, GROUP)], rows.at[0], ssem).wait()
    pltpu.make_async_copy(x_hbm.at[pl.ds(0, GROUP)], rows.at[1], ssem).wait()

    plsc.subcore_barrier()
    pltpu.sync_copy(acc.at[pl.ds(s * rpt, rpt)],
                    out_hbm.at[c, pl.ds(s * rpt, rpt)])


@functools.partial(
    pl.kernel,
    out_type=jax.ShapeDtypeStruct((NC, KP, D), jnp.float32),
    mesh=_MESH,
    compiler_params=_SC_PARAMS,
    scratch_types=[
        pltpu.VMEM((KPW,), jnp.int32),
        pltpu.VMEM((NC, KPW, D), jnp.float32),
        pltpu.SemaphoreType.DMA,
    ],
)
def _sc_gather(aggs_hbm, mask_hbm, out_hbm, midx, vals, sem):
    c = lax.axis_index("c")
    s = lax.axis_index("s")
    wid = c * NS + s
    base = wid * KPW
    pltpu.sync_copy(mask_hbm.at[pl.ds(base, KPW)], midx)
    descs = []
    for cc in range(NC):
        for j in range(NCH3):
            descs.append(pltpu.async_copy(
                aggs_hbm.at[cc].at[midx.at[pl.ds(j * CH3, CH3)]],
                vals.at[cc, pl.ds(j * CH3, CH3)], sem))
    for d in descs:
        d.wait()
    for cc in range(NC):
        pltpu.sync_copy(vals.at[cc], out_hbm.at[cc, pl.ds(base, KPW)])


_BLK2 = 6400


def _mlp_body(sel_ref, w0_ref, b0_ref, w1_ref, b1_ref, w2_ref, b2_ref,
              out_ref):
    t = sel_ref[0] + sel_ref[1]
    deg = jnp.maximum(t[:, DEG_COL:DEG_COL + 1], 1.0)
    sm = t / deg
    h = jnp.maximum(jnp.dot(sm, w0_ref[:], preferred_element_type=jnp.float32)
                    + b0_ref[:], 0.0)
    h = jnp.maximum(jnp.dot(h, w1_ref[:], preferred_element_type=jnp.float32)
                    + b1_ref[:], 0.0)
    lo = jnp.dot(h, w2_ref[:], preferred_element_type=jnp.float32) + b2_ref[:]
    out_ref[:] = lo + jnp.zeros((1, 8), jnp.float32)


def _mlp_head(sel, w0, b0, w1, b1, w2, b2):
    full = lambda shape: pl.BlockSpec(shape, lambda i: tuple(0 for _ in shape))
    return pl.pallas_call(
        _mlp_body,
        grid=(KP // _BLK2,),
        in_specs=[
            pl.BlockSpec((2, _BLK2, D), lambda i: (0, i, 0)),
            full((D, 16)), full((1, 16)),
            full((16, 24)), full((1, 24)),
            full((24, 1)), full((1, 1)),
        ],
        out_specs=pl.BlockSpec((_BLK2, 8), lambda i: (i, 0)),
        out_shape=jax.ShapeDtypeStruct((KP, 8), jnp.float32),
    )(sel, w0, b0, w1, b1, w2, b2)


_R4 = KP // 128


def _softmax_body(sel_ref, out_ref):
    x = sel_ref[:]
    flat_idx = (lax.broadcasted_iota(jnp.int32, (_R4, 128), 0) * 128
                + lax.broadcasted_iota(jnp.int32, (_R4, 128), 1))
    valid = flat_idx < K
    neg = jnp.full_like(x, -jnp.inf)
    m = jnp.max(jnp.where(valid, x, neg))
    e = jnp.where(valid, jnp.exp(x - m), 0.0)
    out_ref[:] = e / jnp.sum(e)


def _softmax(sel2):
    return pl.pallas_call(
        _softmax_body,
        out_shape=jax.ShapeDtypeStruct((_R4, 128), jnp.float32),
    )(sel2)


def kernel(edge_index, real_features, cat_features, mask, emb_table,
           W_gcn, b_gcn, W1, b1, W2, b2):
    x_aug = _build_x(real_features, cat_features, emb_table)

    zeros = jnp.zeros((NS, N // NS, D), jnp.float32)
    aggs = _sc_aggregate(edge_index, x_aug, zeros)

    mask_p = jnp.concatenate([mask, jnp.zeros((KP - K,), jnp.int32)])
    sel = _sc_gather(aggs, mask_p)

    w0 = jnp.zeros((D, 16), jnp.float32).at[:DEG_COL].set(W_gcn)
    logits = _mlp_head(sel, w0, b_gcn.reshape(1, 16),
                       W1, b1.reshape(1, 24), W2, b2.reshape(1, 1))

    probs = _softmax(logits[:, 0].reshape(_R4, 128))
    return probs.reshape(-1)[:K]
